# baseline (speedup 1.0000x reference)
import jax
import jax.numpy as jnp
from jax import lax
from jax.experimental import pallas as pl
from jax.experimental.pallas import tpu as pltpu
from jax.experimental.pallas import tpu_sc as plsc

ROWS = 100000
COLS = 64
NIDX = 16384
NC = 2
NS = 16
PHASES = 5
PIECE = ROWS // (NC * PHASES)
TROWS = 624
REM = PIECE - NS * TROWS
IDX_PER_TILE = NIDX // NS
CH = 128
PAD = CH + 16
TRASH = 16


def _body(x_hbm, idx_hbm, src_hbm, alpha_hbm, out_hbm,
          idxbuf, selpos, selloc, poschunk, locchunk, srcbuf, alphabuf,
          piece0, piece1, sem_in, sem_g, sem_out0, sem_out1):
    c = lax.axis_index("c")
    t = lax.axis_index("s")
    lanes = lax.iota(jnp.int32, 16)

    pltpu.sync_copy(idx_hbm.at[pl.ds(t * IDX_PER_TILE, IDX_PER_TILE)], idxbuf)
    pltpu.sync_copy(alpha_hbm, alphabuf)
    alpha_v = alphabuf[...]
    alpha_s = alpha_v[0]

    pieces = (piece0, piece1)
    out_sems = (sem_out0, sem_out1)

    def phase_base(p):
        return (c * PHASES + p) * PIECE

    def issue_in(p):
        buf = pieces[p % 2]
        row0 = pl.multiple_of(phase_base(p) + t * TROWS, 8)
        return pltpu.async_copy(x_hbm.at[pl.ds(row0, TROWS), :],
                                buf.at[pl.ds(t * TROWS, TROWS), :], sem_in)

    h_in = issue_in(0)
    h_out = [None] * PHASES

    for p in range(PHASES):
        buf = pieces[p % 2]
        base = phase_base(p)

        def scan_step(j, cnt):
            v = idxbuf[pl.ds(j * 16, 16)]
            m = (v >= base) & (v < base + PIECE)
            plsc.store_compressed(selloc.at[pl.ds(cnt, 16)], v - base, mask=m)
            plsc.store_compressed(selpos.at[pl.ds(cnt, 16)],
                                  t * IDX_PER_TILE + j * 16 + lanes, mask=m)
            return cnt + jnp.sum(m.astype(jnp.int32))
        cnt = lax.fori_loop(0, IDX_PER_TILE // 16, scan_step, jnp.int32(0))

        for k in range(CH // 16 + 1):
            selloc[pl.ds(cnt + k * 16, 16)] = PIECE + lanes
            selpos[pl.ds(cnt + k * 16, 16)] = lanes

        h_in.wait()

        @pl.when(t == 0)
        def _():
            rem0 = pl.multiple_of(base + NS * TROWS, 8)
            pltpu.sync_copy(x_hbm.at[pl.ds(rem0, REM), :],
                            buf.at[pl.ds(NS * TROWS, REM), :])

        plsc.subcore_barrier()

        nchunks = (cnt + (CH - 1)) // CH

        def chunk_step(ci, carry):
            for k in range(CH // 16):
                poschunk[pl.ds(k * 16, 16)] = selpos[pl.ds(ci * CH + k * 16, 16)]
                locchunk[pl.ds(k * 16, 16)] = selloc[pl.ds(ci * CH + k * 16, 16)]
            pltpu.async_copy(src_hbm.at[poschunk], srcbuf, sem_g).wait()

            @pl.when(alpha_s != 1.0)
            def _():
                def scale_row(r, carry2):
                    for q in range(COLS // 16):
                        srcbuf[r, pl.ds(q * 16, 16)] = (
                            srcbuf[r, pl.ds(q * 16, 16)] * alpha_v)
                    return carry2
                lax.fori_loop(0, CH, scale_row, jnp.int32(0))

            pltpu.sync_copy(srcbuf, buf.at[locchunk], add=True)
            return carry
        lax.fori_loop(0, nchunks, chunk_step, jnp.int32(0))

        plsc.subcore_barrier()

        row0 = pl.multiple_of(base + t * TROWS, 8)
        h_out[p] = pltpu.async_copy(buf.at[pl.ds(t * TROWS, TROWS), :],
                                    out_hbm.at[pl.ds(row0, TROWS), :],
                                    out_sems[p % 2])

        @pl.when(t == 0)
        def _():
            rem0 = pl.multiple_of(base + NS * TROWS, 8)
            pltpu.sync_copy(buf.at[pl.ds(NS * TROWS, REM), :],
                            out_hbm.at[pl.ds(rem0, REM), :])

        if p + 1 < PHASES:
            if p >= 1:
                h_out[p - 1].wait()
            h_in = issue_in(p + 1)

    h_out[PHASES - 2].wait()
    h_out[PHASES - 1].wait()


def kernel(x, dim, index, source, alpha):
    idx32 = (index + dim).astype(jnp.int32)
    alpha_arr = jnp.full((16,), alpha, dtype=jnp.float32)

    mesh = plsc.VectorSubcoreMesh(core_axis_name="c", subcore_axis_name="s")
    f = pl.kernel(
        _body,
        mesh=mesh,
        compiler_params=pltpu.CompilerParams(needs_layout_passes=False,
                                             use_tc_tiling_on_sc=False),
        out_type=jax.ShapeDtypeStruct((ROWS, COLS), jnp.float32),
        scratch_types=[
            pltpu.VMEM((IDX_PER_TILE,), jnp.int32),
            pltpu.VMEM((IDX_PER_TILE + PAD,), jnp.int32),
            pltpu.VMEM((IDX_PER_TILE + PAD,), jnp.int32),
            pltpu.VMEM((CH,), jnp.int32),
            pltpu.VMEM((CH,), jnp.int32),
            pltpu.VMEM((CH, COLS), jnp.float32),
            pltpu.VMEM((16,), jnp.float32),
            pltpu.VMEM_SHARED((PIECE + TRASH, COLS), jnp.float32),
            pltpu.VMEM_SHARED((PIECE + TRASH, COLS), jnp.float32),
            pltpu.SemaphoreType.DMA,
            pltpu.SemaphoreType.DMA,
            pltpu.SemaphoreType.DMA,
            pltpu.SemaphoreType.DMA,
        ],
    )
    return f(x, idx32, source, alpha_arr)
